# baseline (speedup 1.0000x reference)
import functools

import jax
import jax.numpy as jnp
from jax import lax
from jax.experimental import pallas as pl
from jax.experimental.pallas import tpu as pltpu
from jax.experimental.pallas import tpu_sc as plsc

TQ = 512


def _knn_kernel(q_ref, k_ref, o_ref, *, n1):
    q = q_ref[0]
    k = k_ref[0]

    qn = jnp.sum(q * q, axis=1, keepdims=True)
    kn = jnp.sum(k * k, axis=0, keepdims=True)

    mm = jnp.dot(q, k)
    d = -2.0 * mm
    d = d + qn
    d = d + kn

    iota = lax.broadcasted_iota(jnp.int32, d.shape, 1)
    idxs = []
    for _ in range(3):
        m = jnp.min(d, axis=1, keepdims=True)
        i = jnp.min(jnp.where(d == m, iota, n1), axis=1, keepdims=True)
        idxs.append(i)
        d = jnp.where(iota == i, jnp.inf, d)

    o_ref[0] = jnp.concatenate(idxs, axis=1)


def _knn_topk3(queries, keys):
    b, n2, c = queries.shape
    n1 = keys.shape[2]
    return pl.pallas_call(
        functools.partial(_knn_kernel, n1=n1),
        grid=(b, n2 // TQ),
        in_specs=[
            pl.BlockSpec((1, TQ, c), lambda i, j: (i, j, 0)),
            pl.BlockSpec((1, c, n1), lambda i, j: (i, 0, 0)),
        ],
        out_specs=pl.BlockSpec((1, TQ, c), lambda i, j: (i, j, 0)),
        out_shape=jax.ShapeDtypeStruct((b, n2, c), jnp.int32),
    )(queries, keys)


def _rsqrt16(x):
    i = plsc.bitcast(x, jnp.int32)
    i = jnp.int32(0x5F3759DF) - (i >> 1)
    y = plsc.bitcast(i, jnp.float32)
    hx = 0.5 * x
    for _ in range(3):
        y = y * (1.5 - (hx * y) * y)
    return y


def _sc_combine(knn_idx, keys_rows, flow_rows, queries):
    b, n2, _ = knn_idx.shape
    n1 = keys_rows.shape[1]
    nw = 32
    per_b = nw // b
    qpw = n2 // per_b
    steps = qpw // 16

    @functools.partial(
        pl.kernel,
        mesh=plsc.VectorSubcoreMesh(core_axis_name="c", subcore_axis_name="s"),
        compiler_params=pltpu.CompilerParams(needs_layout_passes=False),
        out_type=jax.ShapeDtypeStruct((b, n2 * 3), jnp.float32),
        scratch_types=[
            pltpu.VMEM((qpw * 3,), jnp.int32),
            pltpu.VMEM((n1 * 3,), jnp.float32),
            pltpu.VMEM((n1 * 3,), jnp.float32),
            pltpu.VMEM((qpw * 3,), jnp.float32),
            pltpu.VMEM((qpw * 3,), jnp.float32),
        ],
    )
    def sc_body(idx_hbm, keys_hbm, flow_hbm, q_hbm, out_hbm,
                idx_v, keys_v, flow_v, x_v, o_v):
        wid = lax.axis_index("s") * 2 + lax.axis_index("c")
        bi = wid // per_b
        qbase = (wid % per_b) * qpw

        pltpu.sync_copy(idx_hbm.at[bi, pl.ds(qbase * 3, qpw * 3)], idx_v)
        pltpu.sync_copy(keys_hbm.at[bi], keys_v)
        pltpu.sync_copy(flow_hbm.at[bi], flow_v)
        pltpu.sync_copy(q_hbm.at[bi, pl.ds(qbase * 3, qpw * 3)], x_v)

        lane3 = lax.iota(jnp.int32, 16) * 3

        def step(si):
            rows3 = si * 48 + lane3
            j3 = [plsc.load_gather(idx_v, [rows3 + t]) * 3 for t in range(3)]
            qc = [plsc.load_gather(x_v, [rows3 + ci]) for ci in range(3)]
            w = []
            for t in range(3):
                dd = jnp.zeros((16,), jnp.float32)
                for ci in range(3):
                    kc = plsc.load_gather(keys_v, [j3[t] + ci])
                    diff = kc - qc[ci]
                    dd = dd + diff * diff
                w.append(jnp.minimum(_rsqrt16(dd), 1e10))
            norm = w[0] + w[1] + w[2]
            for ci in range(3):
                acc = jnp.zeros((16,), jnp.float32)
                for t in range(3):
                    fc = plsc.load_gather(flow_v, [j3[t] + ci])
                    acc = acc + w[t] * fc
                plsc.store_scatter(o_v, [rows3 + ci], qc[ci] - acc / norm)

        for si in range(steps):
            step(si)

        pltpu.sync_copy(o_v, out_hbm.at[bi, pl.ds(qbase * 3, qpw * 3)])

    out = sc_body(knn_idx.reshape(b, n2 * 3),
                  keys_rows.reshape(b, n1 * 3),
                  flow_rows.reshape(b, n1 * 3),
                  queries.reshape(b, n2 * 3))
    return out.reshape(b, n2, 3)


def kernel(xyz1, xyz2, flow1):
    keys = xyz1 + flow1
    queries = jnp.transpose(xyz2, (0, 2, 1))
    knn_idx = _knn_topk3(queries, keys)
    keys_rows = jnp.transpose(keys, (0, 2, 1))
    flow_rows = jnp.transpose(flow1, (0, 2, 1))
    out = _sc_combine(knn_idx, keys_rows, flow_rows, queries)
    return jnp.transpose(out, (0, 2, 1))


# speedup vs baseline: 1.2329x; 1.2329x over previous
import functools

import jax
import jax.numpy as jnp
from jax import lax
from jax.experimental import pallas as pl
from jax.experimental.pallas import tpu as pltpu
from jax.experimental.pallas import tpu_sc as plsc

TQ = 512


def _knn_kernel(q_ref, k_ref, o_ref, *, n1):
    q = q_ref[0]
    k = k_ref[0]

    qn = jnp.sum(q * q, axis=1, keepdims=True)
    kn = jnp.sum(k * k, axis=0, keepdims=True)

    mm = jnp.dot(q, k)
    d = -2.0 * mm
    d = d + qn
    d = d + kn

    iota = lax.broadcasted_iota(jnp.int32, d.shape, 1)
    idxs = []
    for r in range(3):
        m = jnp.min(d, axis=1, keepdims=True)
        i = jnp.min(jnp.where(d == m, iota, n1), axis=1, keepdims=True)
        idxs.append(i)
        if r < 2:
            d = jnp.where(iota == i, jnp.inf, d)

    o_ref[0] = jnp.concatenate(idxs, axis=1)


def _knn_topk3(queries, keys):
    b, n2, c = queries.shape
    n1 = keys.shape[2]
    return pl.pallas_call(
        functools.partial(_knn_kernel, n1=n1),
        grid=(b, n2 // TQ),
        in_specs=[
            pl.BlockSpec((1, TQ, c), lambda i, j: (i, j, 0)),
            pl.BlockSpec((1, c, n1), lambda i, j: (i, 0, 0)),
        ],
        out_specs=pl.BlockSpec((1, TQ, c), lambda i, j: (i, j, 0)),
        out_shape=jax.ShapeDtypeStruct((b, n2, c), jnp.int32),
    )(queries, keys)


def _rsqrt16(x):
    i = plsc.bitcast(x, jnp.int32)
    i = jnp.int32(0x5F3759DF) - (i >> 1)
    y = plsc.bitcast(i, jnp.float32)
    hx = 0.5 * x
    for _ in range(3):
        y = y * (1.5 - (hx * y) * y)
    return y


def _sc_combine(knn_idx, keys, flow1, xyz2):
    b, n2, _ = knn_idx.shape
    n1 = keys.shape[2]
    nw = 32
    per_b = nw // b
    qpw = n2 // per_b
    steps = qpw // 16

    @functools.partial(
        pl.kernel,
        mesh=plsc.VectorSubcoreMesh(core_axis_name="c", subcore_axis_name="s"),
        compiler_params=pltpu.CompilerParams(needs_layout_passes=False),
        out_type=jax.ShapeDtypeStruct((b, 3 * n2), jnp.float32),
        scratch_types=[
            pltpu.VMEM((qpw * 3,), jnp.int32),
            pltpu.VMEM((n1 * 3,), jnp.float32),
            pltpu.VMEM((n1 * 3,), jnp.float32),
            pltpu.VMEM((qpw,), jnp.float32),
            pltpu.VMEM((qpw,), jnp.float32),
            pltpu.VMEM((qpw,), jnp.float32),
            pltpu.VMEM((qpw,), jnp.float32),
            pltpu.VMEM((qpw,), jnp.float32),
            pltpu.VMEM((qpw,), jnp.float32),
        ],
    )
    def sc_body(idx_hbm, keys_hbm, flow_hbm, q_hbm, out_hbm,
                idx_v, keys_v, flow_v, x0, x1, x2, o0, o1, o2):
        x_v = [x0, x1, x2]
        o_v = [o0, o1, o2]
        wid = lax.axis_index("s") * 2 + lax.axis_index("c")
        bi = wid // per_b
        qbase = (wid % per_b) * qpw

        pltpu.sync_copy(idx_hbm.at[bi, pl.ds(qbase * 3, qpw * 3)], idx_v)
        pltpu.sync_copy(keys_hbm.at[bi], keys_v)
        pltpu.sync_copy(flow_hbm.at[bi], flow_v)
        for ci in range(3):
            pltpu.sync_copy(q_hbm.at[bi, pl.ds(ci * n2 + qbase, qpw)],
                            x_v[ci])

        lane3 = lax.iota(jnp.int32, 16) * 3

        def step(si):
            rows3 = si * 48 + lane3
            sl = pl.ds(si * 16, 16)
            j = [plsc.load_gather(idx_v, [rows3 + t]) for t in range(3)]
            qc = [x_v[ci][sl] for ci in range(3)]
            w = []
            for t in range(3):
                dd = jnp.zeros((16,), jnp.float32)
                for ci in range(3):
                    kc = plsc.load_gather(keys_v, [j[t] + ci * n1])
                    diff = kc - qc[ci]
                    dd = dd + diff * diff
                w.append(jnp.minimum(_rsqrt16(dd), 1e10))
            norm = w[0] + w[1] + w[2]
            for ci in range(3):
                acc = jnp.zeros((16,), jnp.float32)
                for t in range(3):
                    fc = plsc.load_gather(flow_v, [j[t] + ci * n1])
                    acc = acc + w[t] * fc
                o_v[ci][sl] = qc[ci] - acc / norm

        for si in range(steps):
            step(si)

        for ci in range(3):
            pltpu.sync_copy(o_v[ci],
                            out_hbm.at[bi, pl.ds(ci * n2 + qbase, qpw)])

    out = sc_body(knn_idx.reshape(b, n2 * 3),
                  keys.reshape(b, 3 * n1),
                  flow1.reshape(b, 3 * n1),
                  xyz2.reshape(b, 3 * n2))
    return out.reshape(b, 3, n2)


def kernel(xyz1, xyz2, flow1):
    keys = xyz1 + flow1
    queries = jnp.transpose(xyz2, (0, 2, 1))
    knn_idx = _knn_topk3(queries, keys)
    return _sc_combine(knn_idx, keys, flow1, xyz2)


# speedup vs baseline: 1.2390x; 1.0049x over previous
import functools

import jax
import jax.numpy as jnp
from jax import lax
from jax.experimental import pallas as pl
from jax.experimental.pallas import tpu as pltpu
from jax.experimental.pallas import tpu_sc as plsc

TQ = 1024


def _knn_kernel(q_ref, k_ref, o_ref, *, n1):
    q = q_ref[0]
    k = k_ref[0]

    qn = jnp.sum(q * q, axis=1, keepdims=True)
    kn = jnp.sum(k * k, axis=0, keepdims=True)

    mm = jnp.dot(q, k)
    d = -2.0 * mm
    d = d + qn
    d = d + kn

    iota = lax.broadcasted_iota(jnp.int32, d.shape, 1)
    idxs = []
    for r in range(3):
        m = jnp.min(d, axis=1, keepdims=True)
        i = jnp.min(jnp.where(d == m, iota, n1), axis=1, keepdims=True)
        idxs.append(i)
        if r < 2:
            d = jnp.where(iota == i, jnp.inf, d)

    o_ref[0] = jnp.concatenate(idxs, axis=1)


def _knn_topk3(queries, keys):
    b, n2, c = queries.shape
    n1 = keys.shape[2]
    return pl.pallas_call(
        functools.partial(_knn_kernel, n1=n1),
        grid=(b, n2 // TQ),
        in_specs=[
            pl.BlockSpec((1, TQ, c), lambda i, j: (i, j, 0)),
            pl.BlockSpec((1, c, n1), lambda i, j: (i, 0, 0)),
        ],
        out_specs=pl.BlockSpec((1, TQ, c), lambda i, j: (i, j, 0)),
        out_shape=jax.ShapeDtypeStruct((b, n2, c), jnp.int32),
    )(queries, keys)


def _rsqrt16(x):
    i = plsc.bitcast(x, jnp.int32)
    i = jnp.int32(0x5F3759DF) - (i >> 1)
    y = plsc.bitcast(i, jnp.float32)
    hx = 0.5 * x
    for _ in range(3):
        y = y * (1.5 - (hx * y) * y)
    return y


def _sc_combine(knn_idx, keys, flow1, xyz2):
    b, n2, _ = knn_idx.shape
    n1 = keys.shape[2]
    nw = 32
    per_b = nw // b
    qpw = n2 // per_b
    steps = qpw // 16

    @functools.partial(
        pl.kernel,
        mesh=plsc.VectorSubcoreMesh(core_axis_name="c", subcore_axis_name="s"),
        compiler_params=pltpu.CompilerParams(needs_layout_passes=False),
        out_type=jax.ShapeDtypeStruct((b, 3 * n2), jnp.float32),
        scratch_types=[
            pltpu.VMEM((qpw * 3,), jnp.int32),
            pltpu.VMEM((n1 * 3,), jnp.float32),
            pltpu.VMEM((n1 * 3,), jnp.float32),
            pltpu.VMEM((qpw,), jnp.float32),
            pltpu.VMEM((qpw,), jnp.float32),
            pltpu.VMEM((qpw,), jnp.float32),
            pltpu.VMEM((qpw,), jnp.float32),
            pltpu.VMEM((qpw,), jnp.float32),
            pltpu.VMEM((qpw,), jnp.float32),
        ],
    )
    def sc_body(idx_hbm, keys_hbm, flow_hbm, q_hbm, out_hbm,
                idx_v, keys_v, flow_v, x0, x1, x2, o0, o1, o2):
        x_v = [x0, x1, x2]
        o_v = [o0, o1, o2]
        wid = lax.axis_index("s") * 2 + lax.axis_index("c")
        bi = wid // per_b
        qbase = (wid % per_b) * qpw

        pltpu.sync_copy(idx_hbm.at[bi, pl.ds(qbase * 3, qpw * 3)], idx_v)
        pltpu.sync_copy(keys_hbm.at[bi], keys_v)
        pltpu.sync_copy(flow_hbm.at[bi], flow_v)
        for ci in range(3):
            pltpu.sync_copy(q_hbm.at[bi, pl.ds(ci * n2 + qbase, qpw)],
                            x_v[ci])

        lane3 = lax.iota(jnp.int32, 16) * 3

        def step(si):
            rows3 = si * 48 + lane3
            sl = pl.ds(si * 16, 16)
            j = [plsc.load_gather(idx_v, [rows3 + t]) for t in range(3)]
            qc = [x_v[ci][sl] for ci in range(3)]
            w = []
            for t in range(3):
                dd = jnp.zeros((16,), jnp.float32)
                for ci in range(3):
                    kc = plsc.load_gather(keys_v, [j[t] + ci * n1])
                    diff = kc - qc[ci]
                    dd = dd + diff * diff
                w.append(jnp.minimum(_rsqrt16(dd), 1e10))
            norm = w[0] + w[1] + w[2]
            for ci in range(3):
                acc = jnp.zeros((16,), jnp.float32)
                for t in range(3):
                    fc = plsc.load_gather(flow_v, [j[t] + ci * n1])
                    acc = acc + w[t] * fc
                o_v[ci][sl] = qc[ci] - acc / norm

        for si in range(steps):
            step(si)

        for ci in range(3):
            pltpu.sync_copy(o_v[ci],
                            out_hbm.at[bi, pl.ds(ci * n2 + qbase, qpw)])

    out = sc_body(knn_idx.reshape(b, n2 * 3),
                  keys.reshape(b, 3 * n1),
                  flow1.reshape(b, 3 * n1),
                  xyz2.reshape(b, 3 * n2))
    return out.reshape(b, 3, n2)


def kernel(xyz1, xyz2, flow1):
    keys = xyz1 + flow1
    queries = jnp.transpose(xyz2, (0, 2, 1))
    knn_idx = _knn_topk3(queries, keys)
    return _sc_combine(knn_idx, keys, flow1, xyz2)


# speedup vs baseline: 1.2393x; 1.0003x over previous
import functools

import jax
import jax.numpy as jnp
from jax import lax
from jax.experimental import pallas as pl
from jax.experimental.pallas import tpu as pltpu
from jax.experimental.pallas import tpu_sc as plsc

TQ = 1024


def _knn_kernel(q_ref, k_ref, o_ref, *, n1):
    q = q_ref[0]
    k = k_ref[0]

    qn = jnp.sum(q * q, axis=1, keepdims=True)
    kn = jnp.sum(k * k, axis=0, keepdims=True)

    mm = jnp.dot(q, k)
    d = -2.0 * mm
    d = d + qn
    d = d + kn

    iota = lax.broadcasted_iota(jnp.int32, d.shape, 1)
    idxs = []
    for r in range(3):
        m = jnp.min(d, axis=1, keepdims=True)
        i = jnp.min(jnp.where(d == m, iota, n1), axis=1, keepdims=True)
        idxs.append(i)
        if r < 2:
            d = jnp.where(iota == i, jnp.inf, d)

    o_ref[0] = jnp.concatenate(idxs, axis=1)


def _knn_topk3(queries, keys):
    b, n2, c = queries.shape
    n1 = keys.shape[2]
    return pl.pallas_call(
        functools.partial(_knn_kernel, n1=n1),
        grid=(b, n2 // TQ),
        in_specs=[
            pl.BlockSpec((1, TQ, c), lambda i, j: (i, j, 0)),
            pl.BlockSpec((1, c, n1), lambda i, j: (i, 0, 0)),
        ],
        out_specs=pl.BlockSpec((1, TQ, c), lambda i, j: (i, j, 0)),
        out_shape=jax.ShapeDtypeStruct((b, n2, c), jnp.int32),
        compiler_params=pltpu.CompilerParams(
            dimension_semantics=("parallel", "parallel")),
    )(queries, keys)


def _rsqrt16(x):
    i = plsc.bitcast(x, jnp.int32)
    i = jnp.int32(0x5F3759DF) - (i >> 1)
    y = plsc.bitcast(i, jnp.float32)
    hx = 0.5 * x
    for _ in range(3):
        y = y * (1.5 - (hx * y) * y)
    return y


def _sc_combine(knn_idx, keys, flow1, xyz2):
    b, n2, _ = knn_idx.shape
    n1 = keys.shape[2]
    nw = 32
    per_b = nw // b
    qpw = n2 // per_b
    steps = qpw // 16

    @functools.partial(
        pl.kernel,
        mesh=plsc.VectorSubcoreMesh(core_axis_name="c", subcore_axis_name="s"),
        compiler_params=pltpu.CompilerParams(needs_layout_passes=False),
        out_type=jax.ShapeDtypeStruct((b, 3 * n2), jnp.float32),
        scratch_types=[
            pltpu.VMEM((qpw * 3,), jnp.int32),
            pltpu.VMEM((n1 * 3,), jnp.float32),
            pltpu.VMEM((n1 * 3,), jnp.float32),
            pltpu.VMEM((qpw,), jnp.float32),
            pltpu.VMEM((qpw,), jnp.float32),
            pltpu.VMEM((qpw,), jnp.float32),
            pltpu.VMEM((qpw,), jnp.float32),
            pltpu.VMEM((qpw,), jnp.float32),
            pltpu.VMEM((qpw,), jnp.float32),
        ],
    )
    def sc_body(idx_hbm, keys_hbm, flow_hbm, q_hbm, out_hbm,
                idx_v, keys_v, flow_v, x0, x1, x2, o0, o1, o2):
        x_v = [x0, x1, x2]
        o_v = [o0, o1, o2]
        wid = lax.axis_index("s") * 2 + lax.axis_index("c")
        bi = wid // per_b
        qbase = (wid % per_b) * qpw

        pltpu.sync_copy(idx_hbm.at[bi, pl.ds(qbase * 3, qpw * 3)], idx_v)
        pltpu.sync_copy(keys_hbm.at[bi], keys_v)
        pltpu.sync_copy(flow_hbm.at[bi], flow_v)
        for ci in range(3):
            pltpu.sync_copy(q_hbm.at[bi, pl.ds(ci * n2 + qbase, qpw)],
                            x_v[ci])

        lane3 = lax.iota(jnp.int32, 16) * 3

        def step(si):
            rows3 = si * 48 + lane3
            sl = pl.ds(si * 16, 16)
            j = [plsc.load_gather(idx_v, [rows3 + t]) for t in range(3)]
            qc = [x_v[ci][sl] for ci in range(3)]
            w = []
            for t in range(3):
                dd = jnp.zeros((16,), jnp.float32)
                for ci in range(3):
                    kc = plsc.load_gather(keys_v, [j[t] + ci * n1])
                    diff = kc - qc[ci]
                    dd = dd + diff * diff
                w.append(jnp.minimum(_rsqrt16(dd), 1e10))
            norm = w[0] + w[1] + w[2]
            for ci in range(3):
                acc = jnp.zeros((16,), jnp.float32)
                for t in range(3):
                    fc = plsc.load_gather(flow_v, [j[t] + ci * n1])
                    acc = acc + w[t] * fc
                o_v[ci][sl] = qc[ci] - acc / norm

        for si in range(steps):
            step(si)

        for ci in range(3):
            pltpu.sync_copy(o_v[ci],
                            out_hbm.at[bi, pl.ds(ci * n2 + qbase, qpw)])

    out = sc_body(knn_idx.reshape(b, n2 * 3),
                  keys.reshape(b, 3 * n1),
                  flow1.reshape(b, 3 * n1),
                  xyz2.reshape(b, 3 * n2))
    return out.reshape(b, 3, n2)


def kernel(xyz1, xyz2, flow1):
    keys = xyz1 + flow1
    queries = jnp.transpose(xyz2, (0, 2, 1))
    knn_idx = _knn_topk3(queries, keys)
    return _sc_combine(knn_idx, keys, flow1, xyz2)


# speedup vs baseline: 1.4489x; 1.1691x over previous
import functools

import jax
import jax.numpy as jnp
from jax import lax
from jax.experimental import pallas as pl
from jax.experimental.pallas import tpu as pltpu
from jax.experimental.pallas import tpu_sc as plsc

TQ = 1024


def _knn_kernel(q_ref, k_ref, o_ref, *, n1):
    q = q_ref[0]
    k = k_ref[0]

    qn = jnp.sum(q * q, axis=1, keepdims=True)
    kn = jnp.sum(k * k, axis=0, keepdims=True)

    mm = jnp.dot(q, k)
    d = -2.0 * mm
    d = d + qn
    d = d + kn

    iota = lax.broadcasted_iota(jnp.int32, d.shape, 1).astype(jnp.float32)
    idxs = []
    for r in range(3):
        m = jnp.min(d, axis=1, keepdims=True)
        i = jnp.min(jnp.where(d == m, iota, jnp.float32(n1)),
                    axis=1, keepdims=True)
        idxs.append(i.astype(jnp.int32))
        if r < 2:
            d = jnp.where(iota == i, jnp.inf, d)

    o_ref[0] = jnp.concatenate(idxs, axis=1)


def _knn_topk3(queries, keys):
    b, n2, c = queries.shape
    n1 = keys.shape[2]
    return pl.pallas_call(
        functools.partial(_knn_kernel, n1=n1),
        grid=(b, n2 // TQ),
        in_specs=[
            pl.BlockSpec((1, TQ, c), lambda i, j: (i, j, 0)),
            pl.BlockSpec((1, c, n1), lambda i, j: (i, 0, 0)),
        ],
        out_specs=pl.BlockSpec((1, TQ, c), lambda i, j: (i, j, 0)),
        out_shape=jax.ShapeDtypeStruct((b, n2, c), jnp.int32),
        compiler_params=pltpu.CompilerParams(
            dimension_semantics=("parallel", "parallel")),
    )(queries, keys)


def _rsqrt16(x):
    i = plsc.bitcast(x, jnp.int32)
    i = jnp.int32(0x5F3759DF) - (i >> 1)
    y = plsc.bitcast(i, jnp.float32)
    hx = 0.5 * x
    for _ in range(3):
        y = y * (1.5 - (hx * y) * y)
    return y


def _sc_combine(knn_idx, keys, flow1, xyz2):
    b, n2, _ = knn_idx.shape
    n1 = keys.shape[2]
    nw = 32
    per_b = nw // b
    qpw = n2 // per_b
    steps = qpw // 16

    @functools.partial(
        pl.kernel,
        mesh=plsc.VectorSubcoreMesh(core_axis_name="c", subcore_axis_name="s"),
        compiler_params=pltpu.CompilerParams(needs_layout_passes=False),
        out_type=jax.ShapeDtypeStruct((b, 3 * n2), jnp.float32),
        scratch_types=[
            pltpu.VMEM((qpw * 3,), jnp.int32),
            pltpu.VMEM((n1 * 3,), jnp.float32),
            pltpu.VMEM((n1 * 3,), jnp.float32),
            pltpu.VMEM((qpw,), jnp.float32),
            pltpu.VMEM((qpw,), jnp.float32),
            pltpu.VMEM((qpw,), jnp.float32),
            pltpu.VMEM((qpw,), jnp.float32),
            pltpu.VMEM((qpw,), jnp.float32),
            pltpu.VMEM((qpw,), jnp.float32),
        ],
    )
    def sc_body(idx_hbm, keys_hbm, flow_hbm, q_hbm, out_hbm,
                idx_v, keys_v, flow_v, x0, x1, x2, o0, o1, o2):
        x_v = [x0, x1, x2]
        o_v = [o0, o1, o2]
        wid = lax.axis_index("s") * 2 + lax.axis_index("c")
        bi = wid // per_b
        qbase = (wid % per_b) * qpw

        pltpu.sync_copy(idx_hbm.at[bi, pl.ds(qbase * 3, qpw * 3)], idx_v)
        pltpu.sync_copy(keys_hbm.at[bi], keys_v)
        pltpu.sync_copy(flow_hbm.at[bi], flow_v)
        for ci in range(3):
            pltpu.sync_copy(q_hbm.at[bi, pl.ds(ci * n2 + qbase, qpw)],
                            x_v[ci])

        lane3 = lax.iota(jnp.int32, 16) * 3

        def step(si):
            rows3 = si * 48 + lane3
            sl = pl.ds(si * 16, 16)
            j = [plsc.load_gather(idx_v, [rows3 + t]) for t in range(3)]
            qc = [x_v[ci][sl] for ci in range(3)]
            w = []
            for t in range(3):
                dd = jnp.zeros((16,), jnp.float32)
                for ci in range(3):
                    kc = plsc.load_gather(keys_v, [j[t] + ci * n1])
                    diff = kc - qc[ci]
                    dd = dd + diff * diff
                w.append(jnp.minimum(_rsqrt16(dd), 1e10))
            norm = w[0] + w[1] + w[2]
            for ci in range(3):
                acc = jnp.zeros((16,), jnp.float32)
                for t in range(3):
                    fc = plsc.load_gather(flow_v, [j[t] + ci * n1])
                    acc = acc + w[t] * fc
                o_v[ci][sl] = qc[ci] - acc / norm

        for si in range(steps):
            step(si)

        for ci in range(3):
            pltpu.sync_copy(o_v[ci],
                            out_hbm.at[bi, pl.ds(ci * n2 + qbase, qpw)])

    out = sc_body(knn_idx.reshape(b, n2 * 3),
                  keys.reshape(b, 3 * n1),
                  flow1.reshape(b, 3 * n1),
                  xyz2.reshape(b, 3 * n2))
    return out.reshape(b, 3, n2)


def kernel(xyz1, xyz2, flow1):
    keys = xyz1 + flow1
    queries = jnp.transpose(xyz2, (0, 2, 1))
    knn_idx = _knn_topk3(queries, keys)
    return _sc_combine(knn_idx, keys, flow1, xyz2)
